# baseline (speedup 1.0000x reference)
import functools

import jax
import jax.numpy as jnp
from jax import lax
from jax.experimental import pallas as pl
from jax.experimental.pallas import tpu as pltpu
from jax.experimental.pallas import tpu_sc as plsc

N_NODES = 10000
N_EDGES = 320000
D = 128
D_EDGE = 16

NC = 2
NS = 16
NW = NC * NS
E_PER_W = N_EDGES // NW
CHUNK = 80
N_CHUNKS = E_PER_W // CHUNK
N_PAIR = N_CHUNKS // 2

AGG_BE = 1000
AGG_NB = N_EDGES // AGG_BE
AGG_SB = 200
AGG_W = 32


def _proj_body(n_ref, ws_ref, wr_ref, ps_ref, pr_ref):
    x = n_ref[...]
    ps_ref[...] = jnp.dot(x, ws_ref[...], preferred_element_type=jnp.float32)
    pr_ref[...] = jnp.dot(x, wr_ref[...], preferred_element_type=jnp.float32)


def _node_body(n_ref, a_ref, w1_ref, w2_ref, b_ref, o_ref):
    o_ref[...] = jnp.maximum(
        jnp.dot(n_ref[...], w1_ref[...], preferred_element_type=jnp.float32)
        + jnp.dot(a_ref[...], w2_ref[...], preferred_element_type=jnp.float32)
        + b_ref[...],
        0.0,
    )


def _sc_body(ps_hbm, pr_hbm, s_hbm, r_hbm,
             ne_hbm,
             sidx_f, ridx_f,
             ps0, pr0, ps1, pr1, out0, out1,
             sem_s0, sem_r0, sem_s1, sem_r1,
             sem_ne0, sem_ne1):
    c = lax.axis_index("c")
    s = lax.axis_index("s")
    wid = s * NC + c
    edge0 = wid * E_PER_W
    set0, sems0 = (ps0, pr0), (sem_s0, sem_r0)
    set1, sems1 = (ps1, pr1), (sem_s1, sem_r1)

    def mk_in(k, bufs, sems):
        ps_v, pr_v = bufs
        ss, sr = sems
        return (
            pltpu.make_async_copy(
                ps_hbm.at[sidx_f.at[pl.ds(k * CHUNK, CHUNK)]], ps_v, ss),
            pltpu.make_async_copy(
                pr_hbm.at[ridx_f.at[pl.ds(k * CHUNK, CHUNK)]], pr_v, sr),
        )

    def start_in(k, bufs, sems):
        for cp in mk_in(k, bufs, sems):
            cp.start()

    def wait_in(k, bufs, sems):
        for cp in mk_in(k, bufs, sems):
            cp.wait()

    def compute(bufs, out_v):
        ps_v, pr_v = bufs
        def row(i, _):
            for j in range(D // 16):
                sl = pl.ds(j * 16, 16)
                out_v[i, sl] = ps_v[i, sl] + pr_v[i, sl]
            return 0
        lax.fori_loop(0, CHUNK, row, 0)

    def start_ne(k, out_v, sem_ne):
        base = edge0 + k * CHUNK
        pltpu.async_copy(out_v, ne_hbm.at[pl.ds(base, CHUNK)], sem_ne)

    def wait_ne(k, out_v, sem_ne):
        base = edge0 + k * CHUNK
        pltpu.make_async_copy(out_v, ne_hbm.at[pl.ds(base, CHUNK)],
                              sem_ne).wait()

    pltpu.sync_copy(s_hbm.at[pl.ds(edge0, E_PER_W)], sidx_f)
    pltpu.sync_copy(r_hbm.at[pl.ds(edge0, E_PER_W)], ridx_f)

    start_in(0, set0, sems0)

    def pair(p, _):
        k0 = 2 * p
        k1 = 2 * p + 1
        start_in(k1, set1, sems1)

        @pl.when(p > 0)
        def _():
            wait_ne(k0, out0, sem_ne0)
        wait_in(k0, set0, sems0)
        compute(set0, out0)
        start_ne(k0, out0, sem_ne0)
        start_in(k0 + 2, set0, sems0)

        @pl.when(p > 0)
        def _():
            wait_ne(k1, out1, sem_ne1)
        wait_in(k1, set1, sems1)
        compute(set1, out1)
        start_ne(k1, out1, sem_ne1)
        return 0
    lax.fori_loop(0, N_PAIR, pair, 0)

    kt = N_CHUNKS - 1
    wait_ne(kt, out0, sem_ne0)
    wait_ne(kt, out1, sem_ne1)
    wait_in(kt, set0, sems0)
    compute(set0, out0)
    start_ne(kt, out0, sem_ne0)
    wait_ne(kt, out0, sem_ne0)


def _edge_agg_body(e_ref, w_ref, b_ref, t_ref, r_ref, ne_ref, agg_ref):
    i = pl.program_id(0)

    @pl.when(i == 0)
    def _():
        agg_ref[...] = jnp.zeros_like(agg_ref)

    ne = jnp.maximum(
        jnp.dot(e_ref[...], w_ref[...], preferred_element_type=jnp.float32)
        + b_ref[...] + t_ref[...],
        0.0,
    )
    ne_ref[...] = ne
    neb = ne.astype(jnp.bfloat16)

    cols = jax.lax.broadcasted_iota(jnp.int32, (AGG_SB, AGG_W), 1)
    r_full = r_ref[0, 0, :]
    rr_full = r_full[:, None]
    for sub in range(AGG_BE // AGG_SB):
        nb = neb[sub * AGG_SB:(sub + 1) * AGG_SB, :]
        rr = jax.lax.slice(rr_full, (sub * AGG_SB, 0),
                           (sub * AGG_SB + AGG_SB, 1))
        r0 = r_full[sub * AGG_SB]
        rmax = r_full[sub * AGG_SB + AGG_SB - 1]
        w0 = jnp.minimum((r0 // 8) * 8, N_NODES - AGG_W)
        nwin = (rmax - w0) // AGG_W + 1

        def win(k, _):
            lob = w0 + k * AGG_W
            wk = jnp.minimum(lob, N_NODES - AGG_W)
            oh = ((rr - wk == cols) & (rr >= lob)).astype(jnp.bfloat16)
            part = jax.lax.dot_general(
                oh, nb, (((0,), (0,)), ((), ())),
                preferred_element_type=jnp.float32)
            agg_ref[pl.ds(wk, AGG_W), :] += part
            return 0
        lax.fori_loop(0, nwin, win, 0)


@functools.lru_cache(maxsize=None)
def _get_sc_edges():
  return pl.kernel(
    _sc_body,
    out_type=jax.ShapeDtypeStruct((N_EDGES, D), jnp.float32),
    mesh=plsc.VectorSubcoreMesh(core_axis_name="c", subcore_axis_name="s",
                                num_cores=NC, num_subcores=NS),
    scratch_types=(
        [
            pltpu.VMEM((E_PER_W,), jnp.int32),
            pltpu.VMEM((E_PER_W,), jnp.int32),
        ]
        + [pltpu.VMEM((CHUNK, D), jnp.float32)] * 6
        + [pltpu.SemaphoreType.DMA] * 6
    ),
  )


@jax.jit
def _run(nodes, edges, senders, receivers, W_edge, b_edge, W_node, b_node):
    w1 = W_edge[:D_EDGE]
    w_es = W_edge[D_EDGE:D_EDGE + D]
    w_er = W_edge[D_EDGE + D:]
    wn1 = W_node[:D]
    wn2 = W_node[D:]
    be = b_edge.reshape(1, D)
    bn = b_node.reshape(1, D)

    nb = 1000
    ps, pr = pl.pallas_call(
        _proj_body,
        grid=(N_NODES // nb,),
        in_specs=[
            pl.BlockSpec((nb, D), lambda i: (i, 0)),
            pl.BlockSpec((D, D), lambda i: (0, 0)),
            pl.BlockSpec((D, D), lambda i: (0, 0)),
        ],
        out_specs=[
            pl.BlockSpec((nb, D), lambda i: (i, 0)),
            pl.BlockSpec((nb, D), lambda i: (i, 0)),
        ],
        out_shape=[
            jax.ShapeDtypeStruct((N_NODES, D), jnp.float32),
            jax.ShapeDtypeStruct((N_NODES, D), jnp.float32),
        ],
    )(nodes, w_es, w_er)

    t_sum = _get_sc_edges()(ps, pr, senders, receivers)

    r3 = receivers.reshape(AGG_NB, 1, AGG_BE)
    new_edges, agg = pl.pallas_call(
        _edge_agg_body,
        grid=(AGG_NB,),
        in_specs=[
            pl.BlockSpec((AGG_BE, D_EDGE), lambda i: (i, 0)),
            pl.BlockSpec((D_EDGE, D), lambda i: (0, 0)),
            pl.BlockSpec((1, D), lambda i: (0, 0)),
            pl.BlockSpec((AGG_BE, D), lambda i: (i, 0)),
            pl.BlockSpec((1, 1, AGG_BE), lambda i: (i, 0, 0)),
        ],
        out_specs=[
            pl.BlockSpec((AGG_BE, D), lambda i: (i, 0)),
            pl.BlockSpec((N_NODES, D), lambda i: (0, 0)),
        ],
        out_shape=[
            jax.ShapeDtypeStruct((N_EDGES, D), jnp.float32),
            jax.ShapeDtypeStruct((N_NODES, D), jnp.float32),
        ],
    )(edges, w1, be, t_sum, r3)

    new_nodes = pl.pallas_call(
        _node_body,
        grid=(N_NODES // nb,),
        in_specs=[
            pl.BlockSpec((nb, D), lambda i: (i, 0)),
            pl.BlockSpec((nb, D), lambda i: (i, 0)),
            pl.BlockSpec((D, D), lambda i: (0, 0)),
            pl.BlockSpec((D, D), lambda i: (0, 0)),
            pl.BlockSpec((1, D), lambda i: (0, 0)),
        ],
        out_specs=pl.BlockSpec((nb, D), lambda i: (i, 0)),
        out_shape=jax.ShapeDtypeStruct((N_NODES, D), jnp.float32),
    )(nodes, agg, wn1, wn2, bn)

    return new_nodes, new_edges


def kernel(nodes, edges, senders, receivers, W_edge, b_edge, W_node, b_node):
    return _run(nodes, edges, senders, receivers,
                W_edge, b_edge, W_node, b_node)


# speedup vs baseline: 1.6435x; 1.6435x over previous
import functools

import jax
import jax.numpy as jnp
from jax import lax
from jax.experimental import pallas as pl
from jax.experimental.pallas import tpu as pltpu
from jax.experimental.pallas import tpu_sc as plsc

N_NODES = 10000
N_EDGES = 320000
D = 128
D_EDGE = 16

NC = 2
NS = 16
NW = NC * NS
E_PER_W = N_EDGES // NW
CHUNK = 80
N_CHUNKS = E_PER_W // CHUNK
N_PAIR = N_CHUNKS // 2

AGG_BE = 1000
AGG_NB = N_EDGES // AGG_BE
AGG_W = 64


def _proj_body(n_ref, ws_ref, wr_ref, ps_ref, pr_ref):
    x = n_ref[...]
    ps_ref[...] = jnp.dot(x, ws_ref[...], preferred_element_type=jnp.float32)
    pr_ref[...] = jnp.dot(x, wr_ref[...], preferred_element_type=jnp.float32)


def _node_body(n_ref, a_ref, w1_ref, w2_ref, b_ref, o_ref):
    o_ref[...] = jnp.maximum(
        jnp.dot(n_ref[...], w1_ref[...], preferred_element_type=jnp.float32)
        + jnp.dot(a_ref[...], w2_ref[...], preferred_element_type=jnp.float32)
        + b_ref[...],
        0.0,
    )


def _sc_body(ps_hbm, s_hbm,
             gp_hbm,
             sidx_f,
             b0, b1, b2, b3,
             sg0, sg1, sg2, sg3,
             sw0, sw1, sw2, sw3):
    c = lax.axis_index("c")
    s = lax.axis_index("s")
    wid = s * NC + c
    edge0 = wid * E_PER_W
    bufs = (b0, b1, b2, b3)
    gsems = (sg0, sg1, sg2, sg3)
    wsems = (sw0, sw1, sw2, sw3)

    def mk_g(k, j):
        return pltpu.make_async_copy(
            ps_hbm.at[sidx_f.at[pl.ds(k * CHUNK, CHUNK)]], bufs[j], gsems[j])

    def mk_w(k, j):
        base = edge0 + k * CHUNK
        return pltpu.make_async_copy(
            bufs[j], gp_hbm.at[pl.ds(base, CHUNK)], wsems[j])

    pltpu.sync_copy(s_hbm.at[pl.ds(edge0, E_PER_W)], sidx_f)

    for j in range(4):
        mk_g(j, j).start()

    last = N_CHUNKS - 1

    def body(i, _):
        for j in range(4):
            k = 4 * i + j
            mk_g(k, j).wait()
            mk_w(k, j).start()
            k2 = k - 2
            j2 = (j + 2) % 4

            @pl.when(k2 >= 0)
            def _():
                mk_w(k2, j2).wait()

            @pl.when((k2 >= 0) & (k2 + 4 <= last))
            def _():
                mk_g(k2 + 4, j2).start()
        return 0
    lax.fori_loop(0, (N_CHUNKS - 1) // 4, body, 0)

    mk_w(last - 2, 2).wait()
    mk_w(last - 1, 3).wait()
    mk_g(last, 0).wait()
    mk_w(last, 0).start()
    mk_w(last, 0).wait()


def _edge_agg_body(e_ref, w_ref, b_ref, gp_ref, r_ref, pr_ref,
                   ne_ref, agg_ref):
    i = pl.program_id(0)

    @pl.when(i == 0)
    def _():
        agg_ref[...] = jnp.zeros_like(agg_ref)

    pre = (
        jnp.dot(e_ref[...], w_ref[...], preferred_element_type=jnp.float32)
        + b_ref[...] + gp_ref[...]
    )
    r = r_ref[0, 0, :]
    rr = r[:, None]
    cols = jax.lax.broadcasted_iota(jnp.int32, (AGG_BE, AGG_W), 1)
    r0 = r[0]
    rmax = r[AGG_BE - 1]
    w0 = jnp.minimum((r0 // 8) * 8, N_NODES - AGG_W)
    nwin = (rmax - w0) // AGG_W + 1

    def mk_oh(k):
        lob = w0 + k * AGG_W
        wk = jnp.minimum(lob, N_NODES - AGG_W)
        oh = ((rr - wk == cols) & (rr >= lob)).astype(jnp.bfloat16)
        return wk, oh

    def expand(k, acc):
        wk, oh = mk_oh(k)
        prw = pr_ref[pl.ds(wk, AGG_W), :].astype(jnp.bfloat16)
        return acc + jax.lax.dot_general(
            oh, prw, (((1,), (0,)), ((), ())),
            preferred_element_type=jnp.float32)
    ne = jnp.maximum(lax.fori_loop(0, nwin, expand, pre), 0.0)
    ne_ref[...] = ne
    neb = ne.astype(jnp.bfloat16)

    def agg_win(k, _):
        wk, oh = mk_oh(k)
        part = jax.lax.dot_general(
            oh, neb, (((0,), (0,)), ((), ())),
            preferred_element_type=jnp.float32)
        agg_ref[pl.ds(wk, AGG_W), :] += part
        return 0
    lax.fori_loop(0, nwin, agg_win, 0)


@functools.lru_cache(maxsize=None)
def _get_sc_edges():
  return pl.kernel(
    _sc_body,
    out_type=jax.ShapeDtypeStruct((N_EDGES, D), jnp.float32),
    mesh=plsc.VectorSubcoreMesh(core_axis_name="c", subcore_axis_name="s",
                                num_cores=NC, num_subcores=NS),
    compiler_params=pltpu.CompilerParams(use_tc_tiling_on_sc=True),
    scratch_types=(
        [pltpu.VMEM((E_PER_W,), jnp.int32)]
        + [pltpu.VMEM((CHUNK, D), jnp.float32)] * 4
        + [pltpu.SemaphoreType.DMA] * 8
    ),
  )


@jax.jit
def _run(nodes, edges, senders, receivers, W_edge, b_edge, W_node, b_node):
    w1 = W_edge[:D_EDGE]
    w_es = W_edge[D_EDGE:D_EDGE + D]
    w_er = W_edge[D_EDGE + D:]
    wn1 = W_node[:D]
    wn2 = W_node[D:]
    be = b_edge.reshape(1, D)
    bn = b_node.reshape(1, D)

    nb = 1000
    ps, pr = pl.pallas_call(
        _proj_body,
        grid=(N_NODES // nb,),
        in_specs=[
            pl.BlockSpec((nb, D), lambda i: (i, 0)),
            pl.BlockSpec((D, D), lambda i: (0, 0)),
            pl.BlockSpec((D, D), lambda i: (0, 0)),
        ],
        out_specs=[
            pl.BlockSpec((nb, D), lambda i: (i, 0)),
            pl.BlockSpec((nb, D), lambda i: (i, 0)),
        ],
        out_shape=[
            jax.ShapeDtypeStruct((N_NODES, D), jnp.float32),
            jax.ShapeDtypeStruct((N_NODES, D), jnp.float32),
        ],
    )(nodes, w_es, w_er)

    gps = _get_sc_edges()(ps, senders)

    r3 = receivers.reshape(AGG_NB, 1, AGG_BE)
    new_edges, agg = pl.pallas_call(
        _edge_agg_body,
        grid=(AGG_NB,),
        in_specs=[
            pl.BlockSpec((AGG_BE, D_EDGE), lambda i: (i, 0)),
            pl.BlockSpec((D_EDGE, D), lambda i: (0, 0)),
            pl.BlockSpec((1, D), lambda i: (0, 0)),
            pl.BlockSpec((AGG_BE, D), lambda i: (i, 0)),
            pl.BlockSpec((1, 1, AGG_BE), lambda i: (i, 0, 0)),
            pl.BlockSpec((N_NODES, D), lambda i: (0, 0)),
        ],
        out_specs=[
            pl.BlockSpec((AGG_BE, D), lambda i: (i, 0)),
            pl.BlockSpec((N_NODES, D), lambda i: (0, 0)),
        ],
        out_shape=[
            jax.ShapeDtypeStruct((N_EDGES, D), jnp.float32),
            jax.ShapeDtypeStruct((N_NODES, D), jnp.float32),
        ],
    )(edges, w1, be, gps, r3, pr)

    new_nodes = pl.pallas_call(
        _node_body,
        grid=(N_NODES // nb,),
        in_specs=[
            pl.BlockSpec((nb, D), lambda i: (i, 0)),
            pl.BlockSpec((nb, D), lambda i: (i, 0)),
            pl.BlockSpec((D, D), lambda i: (0, 0)),
            pl.BlockSpec((D, D), lambda i: (0, 0)),
            pl.BlockSpec((1, D), lambda i: (0, 0)),
        ],
        out_specs=pl.BlockSpec((nb, D), lambda i: (i, 0)),
        out_shape=jax.ShapeDtypeStruct((N_NODES, D), jnp.float32),
    )(nodes, agg, wn1, wn2, bn)

    return new_nodes, new_edges


def kernel(nodes, edges, senders, receivers, W_edge, b_edge, W_node, b_node):
    return _run(nodes, edges, senders, receivers,
                W_edge, b_edge, W_node, b_node)


# speedup vs baseline: 2.0565x; 1.2513x over previous
import functools

import jax
import jax.numpy as jnp
from jax import lax
from jax.experimental import pallas as pl
from jax.experimental.pallas import tpu as pltpu
from jax.experimental.pallas import tpu_sc as plsc

N_NODES = 10000
N_EDGES = 320000
D = 128
D_EDGE = 16

NC = 2
NS = 16
NW = NC * NS
E_PER_W = N_EDGES // NW
CHUNK = 80
N_CHUNKS = E_PER_W // CHUNK
N_PAIR = N_CHUNKS // 2

AGG_BE = 1280
AGG_NB = N_EDGES // AGG_BE
AGG_W = 64


def _proj_body(n_ref, ws_ref, wr_ref, ps_ref, pr_ref):
    x = n_ref[...]
    ps_ref[...] = jnp.dot(x, ws_ref[...], preferred_element_type=jnp.float32)
    pr_ref[...] = jnp.dot(x, wr_ref[...], preferred_element_type=jnp.float32)


def _node_body(n_ref, a_ref, w1_ref, w2_ref, b_ref, o_ref):
    o_ref[...] = jnp.maximum(
        jnp.dot(n_ref[...], w1_ref[...], preferred_element_type=jnp.float32)
        + jnp.dot(a_ref[...], w2_ref[...], preferred_element_type=jnp.float32)
        + b_ref[...],
        0.0,
    )


def _sc_body(ps_hbm, s_hbm,
             gp_hbm,
             sidx_f,
             b0, b1, b2, b3,
             sg0, sg1, sg2, sg3,
             sw0, sw1, sw2, sw3):
    c = lax.axis_index("c")
    s = lax.axis_index("s")
    wid = s * NC + c
    edge0 = wid * E_PER_W
    bufs = (b0, b1, b2, b3)
    gsems = (sg0, sg1, sg2, sg3)
    wsems = (sw0, sw1, sw2, sw3)

    def mk_g(k, j):
        return pltpu.make_async_copy(
            ps_hbm.at[sidx_f.at[pl.ds(k * CHUNK, CHUNK)]], bufs[j], gsems[j])

    def mk_w(k, j):
        base = edge0 + k * CHUNK
        return pltpu.make_async_copy(
            bufs[j], gp_hbm.at[pl.ds(base, CHUNK)], wsems[j])

    pltpu.sync_copy(s_hbm.at[pl.ds(edge0, E_PER_W)], sidx_f)

    for j in range(4):
        mk_g(j, j).start()

    last = N_CHUNKS - 1

    def body(i, _):
        for j in range(4):
            k = 4 * i + j
            mk_g(k, j).wait()
            mk_w(k, j).start()
            k2 = k - 2
            j2 = (j + 2) % 4

            @pl.when(k2 >= 0)
            def _():
                mk_w(k2, j2).wait()

            @pl.when((k2 >= 0) & (k2 + 4 <= last))
            def _():
                mk_g(k2 + 4, j2).start()
        return 0
    lax.fori_loop(0, (N_CHUNKS - 1) // 4, body, 0)

    mk_w(last - 2, 2).wait()
    mk_w(last - 1, 3).wait()
    mk_g(last, 0).wait()
    mk_w(last, 0).start()
    mk_w(last, 0).wait()


def _edge_agg_body(e_ref, w_ref, b_ref, gp_ref, r_ref, pr_ref,
                   ne_ref, agg_ref):
    i = pl.program_id(0)

    @pl.when(i == 0)
    def _():
        agg_ref[...] = jnp.zeros_like(agg_ref)

    pre = (
        jax.lax.dot_general(e_ref[...], w_ref[...], (((0,), (0,)), ((), ())),
                            preferred_element_type=jnp.float32)
        + b_ref[...] + gp_ref[...]
    )
    r = r_ref[0, 0, :]
    rr = r[:, None]
    cols = jax.lax.broadcasted_iota(jnp.int32, (AGG_BE, AGG_W), 1)
    r0 = r[0]
    rmax = r[AGG_BE - 1]
    w0 = jnp.minimum((r0 // 8) * 8, N_NODES - AGG_W)
    nwin = (rmax - w0) // AGG_W + 1

    def mk_oh(k):
        lob = w0 + k * AGG_W
        wk = jnp.minimum(lob, N_NODES - AGG_W)
        oh = ((rr - wk == cols) & (rr >= lob)).astype(jnp.bfloat16)
        return wk, oh

    def expand(k, acc):
        wk, oh = mk_oh(k)
        prw = pr_ref[pl.ds(wk, AGG_W), :].astype(jnp.bfloat16)
        return acc + jax.lax.dot_general(
            oh, prw, (((1,), (0,)), ((), ())),
            preferred_element_type=jnp.float32)
    ne = jnp.maximum(lax.fori_loop(0, nwin, expand, pre), 0.0)
    ne_ref[...] = ne
    neb = ne.astype(jnp.bfloat16)

    def agg_win(k, _):
        wk, oh = mk_oh(k)
        part = jax.lax.dot_general(
            oh, neb, (((0,), (0,)), ((), ())),
            preferred_element_type=jnp.float32)
        agg_ref[pl.ds(wk, AGG_W), :] += part
        return 0
    lax.fori_loop(0, nwin, agg_win, 0)


@functools.lru_cache(maxsize=None)
def _get_sc_edges():
  return pl.kernel(
    _sc_body,
    out_type=jax.ShapeDtypeStruct((N_EDGES, D), jnp.float32),
    mesh=plsc.VectorSubcoreMesh(core_axis_name="c", subcore_axis_name="s",
                                num_cores=NC, num_subcores=NS),
    compiler_params=pltpu.CompilerParams(use_tc_tiling_on_sc=True),
    scratch_types=(
        [pltpu.VMEM((E_PER_W,), jnp.int32)]
        + [pltpu.VMEM((CHUNK, D), jnp.float32)] * 4
        + [pltpu.SemaphoreType.DMA] * 8
    ),
  )


@jax.jit
def _run(nodes, edges, senders, receivers, W_edge, b_edge, W_node, b_node):
    w1 = W_edge[:D_EDGE]
    w_es = W_edge[D_EDGE:D_EDGE + D]
    w_er = W_edge[D_EDGE + D:]
    wn1 = W_node[:D]
    wn2 = W_node[D:]
    be = b_edge.reshape(1, D)
    bn = b_node.reshape(1, D)

    nb = 1000
    ps, pr = pl.pallas_call(
        _proj_body,
        grid=(N_NODES // nb,),
        in_specs=[
            pl.BlockSpec((nb, D), lambda i: (i, 0)),
            pl.BlockSpec((D, D), lambda i: (0, 0)),
            pl.BlockSpec((D, D), lambda i: (0, 0)),
        ],
        out_specs=[
            pl.BlockSpec((nb, D), lambda i: (i, 0)),
            pl.BlockSpec((nb, D), lambda i: (i, 0)),
        ],
        out_shape=[
            jax.ShapeDtypeStruct((N_NODES, D), jnp.float32),
            jax.ShapeDtypeStruct((N_NODES, D), jnp.float32),
        ],
    )(nodes, w_es, w_er)

    gps = _get_sc_edges()(ps, senders)

    r3 = receivers.reshape(AGG_NB, 1, AGG_BE)
    new_edges, agg = pl.pallas_call(
        _edge_agg_body,
        grid=(AGG_NB,),
        in_specs=[
            pl.BlockSpec((D_EDGE, AGG_BE), lambda i: (0, i)),
            pl.BlockSpec((D_EDGE, D), lambda i: (0, 0)),
            pl.BlockSpec((1, D), lambda i: (0, 0)),
            pl.BlockSpec((AGG_BE, D), lambda i: (i, 0)),
            pl.BlockSpec((1, 1, AGG_BE), lambda i: (i, 0, 0)),
            pl.BlockSpec((N_NODES, D), lambda i: (0, 0)),
        ],
        out_specs=[
            pl.BlockSpec((AGG_BE, D), lambda i: (i, 0)),
            pl.BlockSpec((N_NODES, D), lambda i: (0, 0)),
        ],
        out_shape=[
            jax.ShapeDtypeStruct((N_EDGES, D), jnp.float32),
            jax.ShapeDtypeStruct((N_NODES, D), jnp.float32),
        ],
    )(jnp.swapaxes(edges, 0, 1), w1, be, gps, r3, pr)

    new_nodes = pl.pallas_call(
        _node_body,
        grid=(N_NODES // nb,),
        in_specs=[
            pl.BlockSpec((nb, D), lambda i: (i, 0)),
            pl.BlockSpec((nb, D), lambda i: (i, 0)),
            pl.BlockSpec((D, D), lambda i: (0, 0)),
            pl.BlockSpec((D, D), lambda i: (0, 0)),
            pl.BlockSpec((1, D), lambda i: (0, 0)),
        ],
        out_specs=pl.BlockSpec((nb, D), lambda i: (i, 0)),
        out_shape=jax.ShapeDtypeStruct((N_NODES, D), jnp.float32),
    )(nodes, agg, wn1, wn2, bn)

    return new_nodes, new_edges


def kernel(nodes, edges, senders, receivers, W_edge, b_edge, W_node, b_node):
    return _run(nodes, edges, senders, receivers,
                W_edge, b_edge, W_node, b_node)
